# initial kernel scaffold (unmeasured)
import jax
import jax.numpy as jnp
from jax import lax
from jax.experimental import pallas as pl
from jax.experimental.pallas import tpu as pltpu


def kernel(
    x,
):
    def body(*refs):
        pass

    out_shape = jax.ShapeDtypeStruct(..., jnp.float32)
    return pl.pallas_call(body, out_shape=out_shape)(...)


# baseline (device time: 30472 ns/iter reference)
import jax
import jax.numpy as jnp
from jax import lax
from jax.experimental import pallas as pl
from jax.experimental.pallas import tpu as pltpu


def kernel(x):
    xb = x.astype(jnp.bfloat16)
    m, n = xb.shape

    def body(x_ref, out_ref, send_sem, recv_sem):
        my_x = lax.axis_index("x")
        my_y = lax.axis_index("y")
        y_nbr = (my_x, 1 - my_y)

        barrier_sem = pltpu.get_barrier_semaphore()
        pl.semaphore_signal(
            barrier_sem, inc=1, device_id=y_nbr,
            device_id_type=pl.DeviceIdType.MESH,
        )
        pl.semaphore_wait(barrier_sem, 1)

        rdma = pltpu.make_async_remote_copy(
            src_ref=x_ref,
            dst_ref=out_ref.at[pl.ds(my_y * m, m), :],
            send_sem=send_sem,
            recv_sem=recv_sem,
            device_id=y_nbr,
            device_id_type=pl.DeviceIdType.MESH,
        )
        rdma.start()
        out_ref[pl.ds(my_y * m, m), :] = x_ref[...]
        rdma.wait()

    return pl.pallas_call(
        body,
        out_shape=jax.ShapeDtypeStruct((2 * m, n), jnp.bfloat16),
        in_specs=[pl.BlockSpec(memory_space=pltpu.VMEM)],
        out_specs=pl.BlockSpec(memory_space=pltpu.VMEM),
        scratch_shapes=[
            pltpu.SemaphoreType.DMA,
            pltpu.SemaphoreType.DMA,
        ],
        compiler_params=pltpu.CompilerParams(collective_id=0),
    )(xb)


# device time: 23187 ns/iter; 1.3142x vs baseline; 1.3142x over previous
import jax
import jax.numpy as jnp
from jax import lax
from jax.experimental import pallas as pl
from jax.experimental.pallas import tpu as pltpu

P = 8


def kernel(x):
    m, n = x.shape
    hm = m // 2
    sm = hm // P

    def body(x_ref, out_ref, y_send, y_recv, x_send, x_recv):
        my_x = lax.axis_index("x")
        my_y = lax.axis_index("y")
        y_nbr = (my_x, 1 - my_y)
        x_nbr = (1 - my_x, my_y)

        my_base = my_y * m
        peer_base = (1 - my_y) * m

        send_half = my_base + my_x * hm
        out_ref[pl.ds(send_half, hm), :] = x_ref[
            pl.ds(my_x * hm, hm), :
        ].astype(jnp.bfloat16)

        barrier_sem = pltpu.get_barrier_semaphore()
        for nbr in (y_nbr, x_nbr):
            pl.semaphore_signal(
                barrier_sem, inc=1, device_id=nbr,
                device_id_type=pl.DeviceIdType.MESH,
            )
        pl.semaphore_wait(barrier_sem, 2)

        y_rdmas = []
        for s in range(P):
            row = send_half + s * sm
            r = pltpu.make_async_remote_copy(
                src_ref=out_ref.at[pl.ds(row, sm), :],
                dst_ref=out_ref.at[pl.ds(row, sm), :],
                send_sem=y_send.at[s],
                recv_sem=y_recv.at[s],
                device_id=y_nbr,
                device_id_type=pl.DeviceIdType.MESH,
            )
            r.start()
            y_rdmas.append(r)

        keep_half = my_base + (1 - my_x) * hm
        out_ref[pl.ds(keep_half, hm), :] = x_ref[
            pl.ds((1 - my_x) * hm, hm), :
        ].astype(jnp.bfloat16)

        x_rdmas = []
        for s in range(P):
            y_rdmas[s].wait_recv()
            row = peer_base + my_x * hm + s * sm
            r = pltpu.make_async_remote_copy(
                src_ref=out_ref.at[pl.ds(row, sm), :],
                dst_ref=out_ref.at[pl.ds(row, sm), :],
                send_sem=x_send.at[s],
                recv_sem=x_recv.at[s],
                device_id=x_nbr,
                device_id_type=pl.DeviceIdType.MESH,
            )
            r.start()
            x_rdmas.append(r)

        for s in range(P):
            x_rdmas[s].wait_recv()
        for s in range(P):
            y_rdmas[s].wait_send()
            x_rdmas[s].wait_send()

    return pl.pallas_call(
        body,
        out_shape=jax.ShapeDtypeStruct((2 * m, n), jnp.bfloat16),
        in_specs=[pl.BlockSpec(memory_space=pltpu.VMEM)],
        out_specs=pl.BlockSpec(memory_space=pltpu.VMEM),
        scratch_shapes=[
            pltpu.SemaphoreType.DMA((P,)),
            pltpu.SemaphoreType.DMA((P,)),
            pltpu.SemaphoreType.DMA((P,)),
            pltpu.SemaphoreType.DMA((P,)),
        ],
        compiler_params=pltpu.CompilerParams(collective_id=0),
    )(x)
